# baseline (speedup 1.0000x reference)
import functools

import jax
import jax.numpy as jnp
from jax import lax
from jax.experimental import pallas as pl
from jax.experimental.pallas import tpu as pltpu
from jax.experimental.pallas import tpu_sc as plsc

_E, _TOPK, _H, _DFF = 8, 2, 1024, 4096
_T = 2048
_A = _T * _TOPK
_BLK = 640
_NB = -(-_A // _BLK) + _E
_PAD = _NB * _BLK
_EP = 128
_NF = 2
_FBLK = _DFF // _NF
_CHUNK = 512

_NTILES = 32
_APT = _A // _NTILES
_NDC = 4
_DCH = _APT // _NDC
_NRB = 3
_RT = _T // _NTILES
_CH = 16
_NCC = _RT // _CH

_INV_SQRT2 = 0.7071067811865476


def _router_body(xf_ref, gw_ref, pos_ref, rw_ref, nb_ref, aux_ref):
    xf = xf_ref[...]
    gw = gw_ref[...]
    logits = lax.dot_general(xf, gw, (((1,), (1,)), ((), ())),
                             preferred_element_type=jnp.float32)
    col = lax.broadcasted_iota(jnp.int32, (_T, _EP), 1)
    lm = jnp.where(col < _E, logits, jnp.float32(-1e30))
    mx = jnp.max(lm, axis=1, keepdims=True)
    ex = jnp.exp(lm - mx)
    probs = ex / jnp.sum(ex, axis=1, keepdims=True)

    m1 = jnp.max(probs, axis=1, keepdims=True)
    a1 = jnp.min(jnp.where(probs == m1, col, _EP), axis=1, keepdims=True)
    oh1 = col == a1
    probs2 = jnp.where(oh1, jnp.float32(-1.0), probs)
    m2 = jnp.max(probs2, axis=1, keepdims=True)
    a2 = jnp.min(jnp.where(probs2 == m2, col, _EP), axis=1, keepdims=True)
    oh2 = col == a2

    wsum = m1 + m2
    rw1 = m1 / wsum
    rw2 = m2 / wsum

    oh1f = oh1.astype(jnp.float32)
    oh2f = oh2.astype(jnp.float32)
    counts = jnp.sum(oh1f, axis=0, keepdims=True) + jnp.sum(
        oh2f, axis=0, keepdims=True)
    p_mean = jnp.sum(probs, axis=0, keepdims=True) * (1.0 / _T)
    aux = _E * jnp.sum(counts * (1.0 / _T) * p_mean, axis=(0, 1),
                       keepdims=True)
    aux_ref[...] = jnp.broadcast_to(aux, (8, 128))

    nb_e = jnp.floor((counts + (_BLK - 1.0)) * (1.0 / _BLK))
    r_i = lax.broadcasted_iota(jnp.int32, (_EP, _EP), 0)
    c_i = lax.broadcasted_iota(jnp.int32, (_EP, _EP), 1)
    excl = (r_i < c_i).astype(jnp.float32)
    off = _BLK * lax.dot_general(nb_e, excl, (((1,), (0,)), ((), ())),
                                 preferred_element_type=jnp.float32)
    nb_ref[...] = jnp.broadcast_to(nb_e, (8, 128)).astype(jnp.int32)

    tri = (lax.broadcasted_iota(jnp.int32, (_CHUNK, _CHUNK), 0) >=
           lax.broadcasted_iota(jnp.int32, (_CHUNK, _CHUNK), 1)
           ).astype(jnp.float32)
    carry = jnp.zeros((1, _EP), jnp.float32)
    n_chunks = _A // _CHUNK
    per_k = _T // _CHUNK
    for c in range(n_chunks):
        if c < per_k:
            ohc = oh1f[c * _CHUNK:(c + 1) * _CHUNK]
            rwc = rw1[c * _CHUNK:(c + 1) * _CHUNK]
        else:
            ohc = oh2f[(c - per_k) * _CHUNK:(c - per_k + 1) * _CHUNK]
            rwc = rw2[(c - per_k) * _CHUNK:(c - per_k + 1) * _CHUNK]
        incl = lax.dot_general(tri, ohc, (((1,), (0,)), ((), ())),
                               preferred_element_type=jnp.float32) + carry
        carry = carry + jnp.sum(ohc, axis=0, keepdims=True)
        rank = jnp.sum(incl * ohc, axis=1, keepdims=True) - 1.0
        base = jnp.sum(off * ohc, axis=1, keepdims=True)
        pos_ref[c * _CHUNK:(c + 1) * _CHUNK, :] = (
            base + rank).astype(jnp.int32)
        rw_ref[c * _CHUNK:(c + 1) * _CHUNK, :] = rwc


def _ffn_body(be_ref, xs_ref, w1_ref, w2_ref, ws_ref, y_ref, acc_ref):
    f = pl.program_id(1)

    @pl.when(pl.program_id(0) < be_ref[_NB])
    def _():
        _ffn_compute(be_ref, xs_ref, w1_ref, w2_ref, ws_ref, y_ref, acc_ref)


def _ffn_compute(be_ref, xs_ref, w1_ref, w2_ref, ws_ref, y_ref, acc_ref):
    f = pl.program_id(1)
    xb = xs_ref[...].astype(jnp.bfloat16)
    w1b = w1_ref[0].astype(jnp.bfloat16)
    h = lax.dot_general(xb, w1b, (((1,), (1,)), ((), ())),
                        preferred_element_type=jnp.float32)
    g = 0.5 * h * (1.0 + lax.erf(h * _INV_SQRT2))
    w2b = w2_ref[0].astype(jnp.bfloat16)
    contrib = lax.dot_general(g.astype(jnp.bfloat16), w2b,
                              (((1,), (1,)), ((), ())),
                              preferred_element_type=jnp.float32)
    contrib = contrib * ws_ref[...]

    @pl.when(f == 0)
    def _():
        acc_ref[...] = contrib

    @pl.when(f != 0)
    def _():
        acc_ref[...] = acc_ref[...] + contrib

    @pl.when(f == _NF - 1)
    def _():
        y_ref[...] = acc_ref[...]


def _dispatch_body(pos3_hbm, rw3_hbm, xbf_hbm, xs_hbm, ws_hbm,
                   posv, rwv, tokv, rows0, rows1, rows2,
                   gs0, gs1, gs2, ss0, ss1, ss2, wsem):
    c = lax.axis_index("c")
    s = lax.axis_index("s")
    wid = c * 16 + s
    abase = wid * _APT

    def tok_body(j, carry):
        tokv[pl.ds(j * 16, 16)] = (
            lax.iota(jnp.int32, 16) + ((abase + j * 16) % _T))
        return carry

    lax.fori_loop(0, _APT // 16, tok_body, 0)

    pltpu.sync_copy(pos3_hbm.at[wid], posv)
    pltpu.sync_copy(rw3_hbm.at[wid], rwv)

    rows = [rows0, rows1, rows2]
    gsems = [gs0, gs1, gs2]
    ssems = [ss0, ss1, ss2]
    gds = [None] * _NRB
    sds = [None] * _NRB
    wds = []
    for j in range(_NRB):
        gds[j] = pltpu.async_copy(
            xbf_hbm.at[tokv.at[pl.ds(j * _DCH, _DCH)]], rows[j], gsems[j])
    for j in range(_NDC):
        bb = j % _NRB
        gds[bb].wait()
        sds[bb] = pltpu.async_copy(rows[bb], xs_hbm.at[posv.at[j]], ssems[bb])
        wds.append(pltpu.async_copy(rwv.at[j], ws_hbm.at[posv.at[j]], wsem))
        if j + _NRB < _NDC:
            sds[bb].wait()
            gds[bb] = pltpu.async_copy(
                xbf_hbm.at[tokv.at[pl.ds((j + _NRB) * _DCH, _DCH)]],
                rows[bb], gsems[bb])
    for j in range(_NRB):
        sds[j].wait()
    for d in wds:
        d.wait()


def _combine_body(pos_hbm, y_hbm, out_hbm, p0_v, p1_v,
                  a0_v, b0_v, a1_v, b1_v, sa0, sb0, sa1, sb1):
    c = lax.axis_index("c")
    s = lax.axis_index("s")
    wid = c * 16 + s
    base = wid * _RT
    pltpu.sync_copy(pos_hbm.at[pl.ds(base, _RT)], p0_v)
    pltpu.sync_copy(pos_hbm.at[pl.ds(_T + base, _RT)], p1_v)
    abufs, bbufs = [a0_v, a1_v], [b0_v, b1_v]
    asems, bsems = [sa0, sa1], [sb0, sb1]
    cps = [None, None]
    for k in range(2):
        cps[k] = (
            pltpu.async_copy(y_hbm.at[p0_v.at[pl.ds(k * _CH, _CH)]],
                             abufs[k], asems[k]),
            pltpu.async_copy(y_hbm.at[p1_v.at[pl.ds(k * _CH, _CH)]],
                             bbufs[k], bsems[k]))
    for k in range(_NCC):
        kk = k % 2
        cps[kk][0].wait()
        cps[kk][1].wait()
        a_v, b_v = abufs[kk], bbufs[kk]

        def add_body(r, carry):
            for jj in range(_H // 16):
                sl = pl.ds(jj * 16, 16)
                a_v[r, sl] = a_v[r, sl] + b_v[r, sl]
            return carry

        lax.fori_loop(0, _CH, add_body, 0)
        pltpu.sync_copy(a_v, out_hbm.at[pl.ds(base + k * _CH, _CH)])
        if k + 2 < _NCC:
            cps[kk] = (
                pltpu.async_copy(
                    y_hbm.at[p0_v.at[pl.ds((k + 2) * _CH, _CH)]],
                    abufs[kk], asems[kk]),
                pltpu.async_copy(
                    y_hbm.at[p1_v.at[pl.ds((k + 2) * _CH, _CH)]],
                    bbufs[kk], bsems[kk]))


@functools.lru_cache(maxsize=None)
def _dispatch_call():
    mesh = plsc.VectorSubcoreMesh(core_axis_name="c", subcore_axis_name="s")
    return pl.kernel(
        _dispatch_body,
        out_type=(jax.ShapeDtypeStruct((_PAD, _H), jnp.float32),
                  jax.ShapeDtypeStruct((_PAD,), jnp.float32)),
        mesh=mesh,
        compiler_params=pltpu.CompilerParams(needs_layout_passes=False),
        scratch_types=(
            [pltpu.VMEM((_NDC, _DCH), jnp.int32),
             pltpu.VMEM((_NDC, _DCH), jnp.float32),
             pltpu.VMEM((_APT,), jnp.int32)]
            + [pltpu.VMEM((_DCH, _H), jnp.float32) for _ in range(_NRB)]
            + [pltpu.SemaphoreType.DMA] * 7
        ),
    )


@functools.lru_cache(maxsize=None)
def _combine_call():
    mesh = plsc.VectorSubcoreMesh(core_axis_name="c", subcore_axis_name="s")
    return pl.kernel(
        _combine_body,
        out_type=jax.ShapeDtypeStruct((_T, _H), jnp.float32),
        mesh=mesh,
        compiler_params=pltpu.CompilerParams(needs_layout_passes=False),
        scratch_types=(
            [pltpu.VMEM((_RT,), jnp.int32),
             pltpu.VMEM((_RT,), jnp.int32)]
            + [pltpu.VMEM((_CH, _H), jnp.float32) for _ in range(4)]
            + [pltpu.SemaphoreType.DMA] * 4
        ),
    )


def _router_call(xf, gw_pad):
    return pl.pallas_call(
        _router_body,
        out_shape=(
            jax.ShapeDtypeStruct((_A, 1), jnp.int32),
            jax.ShapeDtypeStruct((_A, 1), jnp.float32),
            jax.ShapeDtypeStruct((8, 128), jnp.int32),
            jax.ShapeDtypeStruct((8, 128), jnp.float32),
        ),
    )(xf, gw_pad)


def _ffn_call(bemap, xs, w1, w2, ws2):
    grid_spec = pltpu.PrefetchScalarGridSpec(
        num_scalar_prefetch=1,
        grid=(_NB, _NF),
        in_specs=[
            pl.BlockSpec((_BLK, _H), lambda b, f, be: (b, 0)),
            pl.BlockSpec((1, _FBLK, _H), lambda b, f, be: (be[b], f, 0)),
            pl.BlockSpec((1, _H, _FBLK), lambda b, f, be: (be[b], 0, f)),
            pl.BlockSpec((_BLK, 1), lambda b, f, be: (b, 0)),
        ],
        out_specs=pl.BlockSpec((_BLK, _H), lambda b, f, be: (b, 0)),
        scratch_shapes=[pltpu.VMEM((_BLK, _H), jnp.float32)],
    )
    return pl.pallas_call(
        _ffn_body,
        grid_spec=grid_spec,
        out_shape=jax.ShapeDtypeStruct((_PAD, _H), jnp.float32),
        compiler_params=pltpu.CompilerParams(
            dimension_semantics=("arbitrary", "arbitrary")),
    )(bemap, xs, w1, w2, ws2)


def kernel(x, gate_w, w1, w2):
    b, s, h = x.shape
    xf = x.reshape(_T, _H)
    gw_pad = jnp.zeros((_EP, _H), jnp.float32).at[:_E].set(gate_w)

    pos, rwa, nb8, aux8 = _router_call(xf, gw_pad)

    nb = nb8[0, :_E]
    cnb = jnp.cumsum(nb)
    total = cnb[_E - 1]
    bi = jnp.minimum(jnp.arange(_NB, dtype=jnp.int32), total - 1)
    blk_e = jnp.sum((bi[:, None] >= cnb[None, :]).astype(jnp.int32),
                    axis=1).astype(jnp.int32)
    bemap = jnp.concatenate([blk_e, total[None]])

    pos_flat = pos.reshape(_A)
    pos3 = pos_flat.reshape(_NTILES, _NDC, _DCH)
    rw3 = rwa.reshape(_NTILES, _NDC, _DCH)
    xs, ws = _dispatch_call()(pos3, rw3, xf)
    y = _ffn_call(bemap, xs, w1, w2, ws.reshape(_PAD, 1))
    out = _combine_call()(pos_flat, y)

    return out.reshape(b, s, h), aux8[0, 0]


# speedup vs baseline: 1.0206x; 1.0206x over previous
import functools

import jax
import jax.numpy as jnp
from jax import lax
from jax.experimental import pallas as pl
from jax.experimental.pallas import tpu as pltpu
from jax.experimental.pallas import tpu_sc as plsc

_E, _TOPK, _H, _DFF = 8, 2, 1024, 4096
_T = 2048
_A = _T * _TOPK
_BLK = 640
_NB = -(-_A // _BLK) + _E
_PAD = _NB * _BLK
_EP = 128
_NF = 2
_FBLK = _DFF // _NF
_CHUNK = 512

_NTILES = 32
_APT = _A // _NTILES
_NDC = 8
_DCH = _APT // _NDC
_NRB = 4
_RT = _T // _NTILES
_CH = 16
_NCC = _RT // _CH

_INV_SQRT2 = 0.7071067811865476


def _router_body(xf_ref, gw_ref, pos_ref, rw_ref, nb_ref, aux_ref):
    xf = xf_ref[...]
    gw = gw_ref[...]
    logits = lax.dot_general(xf, gw, (((1,), (1,)), ((), ())),
                             preferred_element_type=jnp.float32)
    col = lax.broadcasted_iota(jnp.int32, (_T, _EP), 1)
    lm = jnp.where(col < _E, logits, jnp.float32(-1e30))
    mx = jnp.max(lm, axis=1, keepdims=True)
    ex = jnp.exp(lm - mx)
    probs = ex / jnp.sum(ex, axis=1, keepdims=True)

    m1 = jnp.max(probs, axis=1, keepdims=True)
    a1 = jnp.min(jnp.where(probs == m1, col, _EP), axis=1, keepdims=True)
    oh1 = col == a1
    probs2 = jnp.where(oh1, jnp.float32(-1.0), probs)
    m2 = jnp.max(probs2, axis=1, keepdims=True)
    a2 = jnp.min(jnp.where(probs2 == m2, col, _EP), axis=1, keepdims=True)
    oh2 = col == a2

    wsum = m1 + m2
    rw1 = m1 / wsum
    rw2 = m2 / wsum

    oh1f = oh1.astype(jnp.float32)
    oh2f = oh2.astype(jnp.float32)
    counts = jnp.sum(oh1f, axis=0, keepdims=True) + jnp.sum(
        oh2f, axis=0, keepdims=True)
    p_mean = jnp.sum(probs, axis=0, keepdims=True) * (1.0 / _T)
    aux = _E * jnp.sum(counts * (1.0 / _T) * p_mean, axis=(0, 1),
                       keepdims=True)
    aux_ref[...] = jnp.broadcast_to(aux, (8, 128))

    nb_e = jnp.floor((counts + (_BLK - 1.0)) * (1.0 / _BLK))
    r_i = lax.broadcasted_iota(jnp.int32, (_EP, _EP), 0)
    c_i = lax.broadcasted_iota(jnp.int32, (_EP, _EP), 1)
    excl = (r_i < c_i).astype(jnp.float32)
    off = _BLK * lax.dot_general(nb_e, excl, (((1,), (0,)), ((), ())),
                                 preferred_element_type=jnp.float32)
    nb_ref[...] = jnp.broadcast_to(nb_e, (8, 128)).astype(jnp.int32)

    tri = (lax.broadcasted_iota(jnp.int32, (_CHUNK, _CHUNK), 0) >=
           lax.broadcasted_iota(jnp.int32, (_CHUNK, _CHUNK), 1)
           ).astype(jnp.float32)
    carry = jnp.zeros((1, _EP), jnp.float32)
    n_chunks = _A // _CHUNK
    per_k = _T // _CHUNK
    for c in range(n_chunks):
        if c < per_k:
            ohc = oh1f[c * _CHUNK:(c + 1) * _CHUNK]
            rwc = rw1[c * _CHUNK:(c + 1) * _CHUNK]
        else:
            ohc = oh2f[(c - per_k) * _CHUNK:(c - per_k + 1) * _CHUNK]
            rwc = rw2[(c - per_k) * _CHUNK:(c - per_k + 1) * _CHUNK]
        incl = lax.dot_general(tri, ohc, (((1,), (0,)), ((), ())),
                               preferred_element_type=jnp.float32) + carry
        carry = carry + jnp.sum(ohc, axis=0, keepdims=True)
        rank = jnp.sum(incl * ohc, axis=1, keepdims=True) - 1.0
        base = jnp.sum(off * ohc, axis=1, keepdims=True)
        pos_ref[c * _CHUNK:(c + 1) * _CHUNK, :] = (
            base + rank).astype(jnp.int32)
        rw_ref[c * _CHUNK:(c + 1) * _CHUNK, :] = rwc


def _ffn_body(be_ref, xs_ref, w1_ref, w2_ref, ws_ref, y_ref, acc_ref):
    f = pl.program_id(1)

    @pl.when(pl.program_id(0) < be_ref[_NB])
    def _():
        _ffn_compute(be_ref, xs_ref, w1_ref, w2_ref, ws_ref, y_ref, acc_ref)


def _ffn_compute(be_ref, xs_ref, w1_ref, w2_ref, ws_ref, y_ref, acc_ref):
    f = pl.program_id(1)
    xb = xs_ref[...].astype(jnp.bfloat16)
    w1b = w1_ref[0].astype(jnp.bfloat16)
    h = lax.dot_general(xb, w1b, (((1,), (1,)), ((), ())),
                        preferred_element_type=jnp.float32)
    g = 0.5 * h * (1.0 + lax.erf(h * _INV_SQRT2))
    w2b = w2_ref[0].astype(jnp.bfloat16)
    contrib = lax.dot_general(g.astype(jnp.bfloat16), w2b,
                              (((1,), (1,)), ((), ())),
                              preferred_element_type=jnp.float32)
    contrib = contrib * ws_ref[...]

    @pl.when(f == 0)
    def _():
        acc_ref[...] = contrib

    @pl.when(f != 0)
    def _():
        acc_ref[...] = acc_ref[...] + contrib

    @pl.when(f == _NF - 1)
    def _():
        y_ref[...] = acc_ref[...]


def _dispatch_body(pos_hbm, rw_hbm, xf_hbm, xs_hbm, ws_hbm,
                   posv, rwv, rows0, rows1, rows2, rows3,
                   gs0, gs1, gs2, gs3, ss0, ss1, ss2, ss3, wsem):
    c = lax.axis_index("c")
    s = lax.axis_index("s")
    wid = c * 16 + s
    abase = wid * _APT
    pltpu.sync_copy(pos_hbm.at[wid], posv)
    pltpu.sync_copy(rw_hbm.at[wid], rwv)

    def tok_idx(j):
        return lax.iota(jnp.int32, 16) + ((abase + j * _DCH) % _T)

    rows = [rows0, rows1, rows2, rows3]
    gsems = [gs0, gs1, gs2, gs3]
    ssems = [ss0, ss1, ss2, ss3]
    gds = [None] * _NRB
    sds = [None] * _NRB
    wds = []
    for j in range(_NRB):
        gds[j] = pltpu.async_copy(xf_hbm.at[tok_idx(j)], rows[j], gsems[j])
    for j in range(_NDC):
        bb = j % _NRB
        pj = posv[j]
        gds[bb].wait()
        sds[bb] = pltpu.async_copy(rows[bb], xs_hbm.at[pj], ssems[bb])
        wds.append(pltpu.async_copy(rwv.at[j], ws_hbm.at[pj], wsem))
        if j + _NRB < _NDC:
            sds[bb].wait()
            gds[bb] = pltpu.async_copy(
                xf_hbm.at[tok_idx(j + _NRB)], rows[bb], gsems[bb])
    for j in range(_NRB):
        sds[j].wait()
    for d in wds:
        d.wait()


def _combine_body(pos_hbm, y_hbm, out_hbm, p0_v, p1_v,
                  a0_v, b0_v, a1_v, b1_v, sa0, sb0, sa1, sb1):
    c = lax.axis_index("c")
    s = lax.axis_index("s")
    wid = c * 16 + s
    base = wid * _RT
    pltpu.sync_copy(pos_hbm.at[pl.ds(base, _RT)], p0_v)
    pltpu.sync_copy(pos_hbm.at[pl.ds(_T + base, _RT)], p1_v)
    abufs, bbufs = [a0_v, a1_v], [b0_v, b1_v]
    asems, bsems = [sa0, sa1], [sb0, sb1]
    cps = [None, None]
    for k in range(2):
        cps[k] = (
            pltpu.async_copy(y_hbm.at[p0_v.at[pl.ds(k * _CH, _CH)]],
                             abufs[k], asems[k]),
            pltpu.async_copy(y_hbm.at[p1_v.at[pl.ds(k * _CH, _CH)]],
                             bbufs[k], bsems[k]))
    for k in range(_NCC):
        kk = k % 2
        cps[kk][0].wait()
        cps[kk][1].wait()
        a_v, b_v = abufs[kk], bbufs[kk]

        def add_body(r, carry):
            for jj in range(_H // 16):
                sl = pl.ds(jj * 16, 16)
                a_v[r, sl] = a_v[r, sl] + b_v[r, sl]
            return carry

        lax.fori_loop(0, _CH, add_body, 0)
        pltpu.sync_copy(a_v, out_hbm.at[pl.ds(base + k * _CH, _CH)])
        if k + 2 < _NCC:
            cps[kk] = (
                pltpu.async_copy(
                    y_hbm.at[p0_v.at[pl.ds((k + 2) * _CH, _CH)]],
                    abufs[kk], asems[kk]),
                pltpu.async_copy(
                    y_hbm.at[p1_v.at[pl.ds((k + 2) * _CH, _CH)]],
                    bbufs[kk], bsems[kk]))


@functools.lru_cache(maxsize=None)
def _dispatch_call():
    mesh = plsc.VectorSubcoreMesh(core_axis_name="c", subcore_axis_name="s")
    return pl.kernel(
        _dispatch_body,
        out_type=(jax.ShapeDtypeStruct((_PAD, _H), jnp.float32),
                  jax.ShapeDtypeStruct((_PAD,), jnp.float32)),
        mesh=mesh,
        compiler_params=pltpu.CompilerParams(needs_layout_passes=False),
        scratch_types=(
            [pltpu.VMEM((_NDC, _DCH), jnp.int32),
             pltpu.VMEM((_NDC, _DCH), jnp.float32)]
            + [pltpu.VMEM((_DCH, _H), jnp.float32) for _ in range(_NRB)]
            + [pltpu.SemaphoreType.DMA] * 9
        ),
    )


@functools.lru_cache(maxsize=None)
def _combine_call():
    mesh = plsc.VectorSubcoreMesh(core_axis_name="c", subcore_axis_name="s")
    return pl.kernel(
        _combine_body,
        out_type=jax.ShapeDtypeStruct((_T, _H), jnp.float32),
        mesh=mesh,
        compiler_params=pltpu.CompilerParams(needs_layout_passes=False),
        scratch_types=(
            [pltpu.VMEM((_RT,), jnp.int32),
             pltpu.VMEM((_RT,), jnp.int32)]
            + [pltpu.VMEM((_CH, _H), jnp.float32) for _ in range(4)]
            + [pltpu.SemaphoreType.DMA] * 4
        ),
    )


def _router_call(xf, gw_pad):
    return pl.pallas_call(
        _router_body,
        out_shape=(
            jax.ShapeDtypeStruct((_A, 1), jnp.int32),
            jax.ShapeDtypeStruct((_A, 1), jnp.float32),
            jax.ShapeDtypeStruct((8, 128), jnp.int32),
            jax.ShapeDtypeStruct((8, 128), jnp.float32),
        ),
    )(xf, gw_pad)


def _ffn_call(bemap, xs, w1, w2, ws2):
    grid_spec = pltpu.PrefetchScalarGridSpec(
        num_scalar_prefetch=1,
        grid=(_NB, _NF),
        in_specs=[
            pl.BlockSpec((_BLK, _H), lambda b, f, be: (b, 0)),
            pl.BlockSpec((1, _FBLK, _H), lambda b, f, be: (be[b], f, 0)),
            pl.BlockSpec((1, _H, _FBLK), lambda b, f, be: (be[b], 0, f)),
            pl.BlockSpec((_BLK, 1), lambda b, f, be: (b, 0)),
        ],
        out_specs=pl.BlockSpec((_BLK, _H), lambda b, f, be: (b, 0)),
        scratch_shapes=[pltpu.VMEM((_BLK, _H), jnp.float32)],
    )
    return pl.pallas_call(
        _ffn_body,
        grid_spec=grid_spec,
        out_shape=jax.ShapeDtypeStruct((_PAD, _H), jnp.float32),
        compiler_params=pltpu.CompilerParams(
            dimension_semantics=("arbitrary", "arbitrary")),
    )(bemap, xs, w1, w2, ws2)


def kernel(x, gate_w, w1, w2):
    b, s, h = x.shape
    xf = x.reshape(_T, _H)
    gw_pad = jnp.zeros((_EP, _H), jnp.float32).at[:_E].set(gate_w)

    pos, rwa, nb8, aux8 = _router_call(xf, gw_pad)

    nb = nb8[0, :_E]
    cnb = jnp.cumsum(nb)
    total = cnb[_E - 1]
    bi = jnp.minimum(jnp.arange(_NB, dtype=jnp.int32), total - 1)
    blk_e = jnp.sum((bi[:, None] >= cnb[None, :]).astype(jnp.int32),
                    axis=1).astype(jnp.int32)
    bemap = jnp.concatenate([blk_e, total[None]])

    pos_flat = pos.reshape(_A)
    pos3 = pos_flat.reshape(_NTILES, _NDC, _DCH)
    rw3 = rwa.reshape(_NTILES, _NDC, _DCH)
    xs, ws = _dispatch_call()(pos3, rw3, xf)
    y = _ffn_call(bemap, xs, w1, w2, ws.reshape(_PAD, 1))
    out = _combine_call()(pos_flat, y)

    return out.reshape(b, s, h), aux8[0, 0]
